# baseline (speedup 1.0000x reference)
import jax
import jax.numpy as jnp
from jax import lax
from jax.experimental import pallas as pl
from jax.experimental.pallas import tpu as pltpu
from jax.experimental.pallas import tpu_sc as plsc

NSAMPLES = 100000
NDIM = 128
B = 16384
P = 131072

NC = 2
NS = 16
NW = NC * NS
L = 16

RPW = B // NW
PPW = P // NW
CH = 128
NCH_A = RPW // CH
NCH_B = PPW // CH
NG = CH // L


def _loss_body(u_hbm, enc_hbm, sw_hbm, pw_hbm, pa_hbm, pb_hbm, idx_hbm,
               s1_hbm, s2_hbm, lam_hbm, out_hbm,
               idxall, pa_v, pb_v, ua_all, ub_all, a0, a1, b0, b1,
               swv, pwv, s1v, s2v, lamv, accv, sa0, sa1, sb0, sb1):
    wid = lax.axis_index("s") * NC + lax.axis_index("c")
    rbase = wid * RPW
    pbase = wid * PPW

    pltpu.sync_copy(idx_hbm, idxall)
    pltpu.sync_copy(pa_hbm.at[pl.ds(pbase, PPW)], pa_v)
    pltpu.sync_copy(pb_hbm.at[pl.ds(pbase, PPW)], pb_v)
    pltpu.sync_copy(sw_hbm.at[pl.ds(rbase, RPW)], swv)
    pltpu.sync_copy(pw_hbm.at[pl.ds(pbase, PPW)], pwv)
    pltpu.sync_copy(s1_hbm, s1v)
    pltpu.sync_copy(s2_hbm, s2v)
    pltpu.sync_copy(lam_hbm, lamv)

    accv[...] = jnp.zeros((L,), jnp.float32)
    iota16 = lax.iota(jnp.int32, L)
    s1 = s1v[...]
    s2 = s2v[...]
    lam = lamv[...]

    abuf = (a0, a1)
    bbuf = (b0, b1)
    asem = (sa0, sa1)
    bsem = (sb0, sb1)

    def translate(t, _):
        pav = pa_v[pl.ds(t * L, L)]
        ua_all[pl.ds(t * L, L)] = plsc.load_gather(idxall, [pav])
        pbv = pb_v[pl.ds(t * L, L)]
        ub_all[pl.ds(t * L, L)] = plsc.load_gather(idxall, [pbv])

    lax.fori_loop(0, PPW // L, translate, None)

    def row_sums(ar, br):
        def sums_at(g):
            ov = jnp.zeros((L,), jnp.float32)
            for r in range(L):
                row = g * L + r
                acc16 = jnp.zeros((L,), jnp.float32)
                for k in range(NDIM // L):
                    xv = ar[row, pl.ds(k * L, L)]
                    yv = br[row, pl.ds(k * L, L)]
                    df = xv - yv
                    acc16 = acc16 + df * df
                ov = ov + jnp.where(iota16 == r, jnp.sum(acc16), 0.0)
            return ov
        return sums_at

    def issue_a(sub, i):
        da = pltpu.async_copy(
            u_hbm.at[idxall.at[pl.ds(rbase + sub * CH, CH)]], abuf[i], asem[i])
        db = pltpu.async_copy(
            enc_hbm.at[pl.ds(rbase + sub * CH, CH), :], bbuf[i], bsem[i])
        return da, db

    pend = issue_a(0, 0)
    for sub in range(NCH_A):
        i = sub % 2
        nxt = issue_a(sub + 1, 1 - i) if sub + 1 < NCH_A else None
        pend[0].wait()
        pend[1].wait()
        sums = row_sums(abuf[i], bbuf[i])

        def group_a(g, _, sub=sub, sums=sums):
            o1v = sums(g)
            w16 = swv[pl.ds(sub * CH + g * L, L)]
            accv[...] = accv[...] + (s1 * w16 * o1v) / (s1 + o1v)

        lax.fori_loop(0, NG, group_a, None)
        pend = nxt

    def issue_b(c, i):
        pltpu.async_copy(u_hbm.at[ua_all.at[pl.ds(c * CH, CH)]],
                         abuf[i], asem[i])
        pltpu.async_copy(u_hbm.at[ub_all.at[pl.ds(c * CH, CH)]],
                         bbuf[i], bsem[i])

    def wait_b(i):
        pltpu.make_async_copy(u_hbm.at[ua_all.at[pl.ds(0, CH)]],
                              abuf[i], asem[i]).wait()
        pltpu.make_async_copy(u_hbm.at[ub_all.at[pl.ds(0, CH)]],
                              bbuf[i], bsem[i]).wait()

    def compute_b(c, i):
        sums = row_sums(abuf[i], bbuf[i])

        def group_b(g, _):
            o2v = sums(g)
            pw16 = pwv[pl.ds(c * CH + g * L, L)]
            accv[...] = accv[...] + (lam * s2 * pw16 * o2v) / (s2 + o2v)

        lax.fori_loop(0, NG, group_b, None)

    def chunk_pair_b(t, _):
        c0 = 2 * t
        compute_b(c0, 0)
        compute_b(c0 + 1, 1)

    lax.fori_loop(0, NCH_B // 2, chunk_pair_b, None)

    pltpu.sync_copy(accv, out_hbm.at[pl.ds(wid * L, L)])


_loss_kernel = pl.kernel(
    _loss_body,
    out_type=jax.ShapeDtypeStruct((NW * L,), jnp.float32),
    mesh=plsc.VectorSubcoreMesh(core_axis_name="c", subcore_axis_name="s"),
    compiler_params=pltpu.CompilerParams(needs_layout_passes=False),
    scratch_types=[
        pltpu.VMEM((B,), jnp.int32),
        pltpu.VMEM((PPW,), jnp.int32),
        pltpu.VMEM((PPW,), jnp.int32),
        pltpu.VMEM((PPW,), jnp.int32),
        pltpu.VMEM((PPW,), jnp.int32),
        pltpu.VMEM((CH, NDIM), jnp.float32),
        pltpu.VMEM((CH, NDIM), jnp.float32),
        pltpu.VMEM((CH, NDIM), jnp.float32),
        pltpu.VMEM((CH, NDIM), jnp.float32),
        pltpu.VMEM((RPW,), jnp.float32),
        pltpu.VMEM((PPW,), jnp.float32),
        pltpu.VMEM((L,), jnp.float32),
        pltpu.VMEM((L,), jnp.float32),
        pltpu.VMEM((L,), jnp.float32),
        pltpu.VMEM((L,), jnp.float32),
        pltpu.SemaphoreType.DMA,
        pltpu.SemaphoreType.DMA,
        pltpu.SemaphoreType.DMA,
        pltpu.SemaphoreType.DMA,
    ],
)


def kernel(enc_out, sampweights, pairweights, pairs, index, _sigma1, _sigma2,
           _lambda, U):
    pa = pairs[:, 0].astype(jnp.int32)
    pb = pairs[:, 1].astype(jnp.int32)
    idx = index.astype(jnp.int32)
    s1v = jnp.full((L,), _sigma1, jnp.float32)
    s2v = jnp.full((L,), _sigma2, jnp.float32)
    lamv = jnp.full((L,), _lambda, jnp.float32)
    partials = _loss_kernel(U, enc_out, sampweights, pairweights, pa, pb, idx,
                            s1v, s2v, lamv)
    return jnp.sum(partials) / (enc_out.shape[0] * enc_out.shape[1])


# speedup vs baseline: 1.5078x; 1.5078x over previous
import jax
import jax.numpy as jnp
from jax import lax
from jax.experimental import pallas as pl
from jax.experimental.pallas import tpu as pltpu
from jax.experimental.pallas import tpu_sc as plsc

NSAMPLES = 100000
NDIM = 128
B = 16384
P = 131072

NC = 2
NS = 16
NW = NC * NS
L = 16

RPW = B // NW
PPW = P // NW
CH = 128
NCH_A = RPW // CH
NCH_B = PPW // CH
NG = CH // L


def _loss_body(u_hbm, enc_hbm, sw_hbm, pw_hbm, pa_hbm, pb_hbm, idx_hbm,
               s1_hbm, s2_hbm, lam_hbm, out_hbm,
               idxall, pa_v, pb_v, ua_all, ub_all, a0, a1, b0, b1,
               swv, pwv, s1v, s2v, lamv, accv, sa0, sa1, sb0, sb1):
    wid = lax.axis_index("s") * NC + lax.axis_index("c")
    rbase = wid * RPW
    pbase = wid * PPW

    pltpu.sync_copy(idx_hbm, idxall)
    pltpu.sync_copy(pa_hbm.at[pl.ds(pbase, PPW)], pa_v)
    pltpu.sync_copy(pb_hbm.at[pl.ds(pbase, PPW)], pb_v)
    pltpu.sync_copy(sw_hbm.at[pl.ds(rbase, RPW)], swv)
    pltpu.sync_copy(pw_hbm.at[pl.ds(pbase, PPW)], pwv)
    pltpu.sync_copy(s1_hbm, s1v)
    pltpu.sync_copy(s2_hbm, s2v)
    pltpu.sync_copy(lam_hbm, lamv)

    accv[...] = jnp.zeros((L,), jnp.float32)
    iota16 = lax.iota(jnp.int32, L)
    s1 = s1v[...]
    s2 = s2v[...]
    lam = lamv[...]

    abuf = (a0, a1)
    bbuf = (b0, b1)
    asem = (sa0, sa1)
    bsem = (sb0, sb1)

    def translate(t, _):
        pav = pa_v[pl.ds(t * L, L)]
        ua_all[pl.ds(t * L, L)] = plsc.load_gather(idxall, [pav])
        pbv = pb_v[pl.ds(t * L, L)]
        ub_all[pl.ds(t * L, L)] = plsc.load_gather(idxall, [pbv])

    lax.fori_loop(0, PPW // L, translate, None)

    def row_sums(ar, br):
        def sums_at(g):
            def quad(q, ov):
                for r4 in range(4):
                    lane = q * 4 + r4
                    row = g * L + lane
                    acc16 = jnp.zeros((L,), jnp.float32)
                    for k in range(NDIM // L):
                        xv = ar[row, pl.ds(k * L, L)]
                        yv = br[row, pl.ds(k * L, L)]
                        df = xv - yv
                        acc16 = acc16 + df * df
                    ov = ov + jnp.where(iota16 == lane, jnp.sum(acc16), 0.0)
                return ov
            return lax.fori_loop(0, 4, quad, jnp.zeros((L,), jnp.float32))
        return sums_at

    def issue_a(sub, i):
        da = pltpu.async_copy(
            u_hbm.at[idxall.at[pl.ds(rbase + sub * CH, CH)]], abuf[i], asem[i])
        db = pltpu.async_copy(
            enc_hbm.at[pl.ds(rbase + sub * CH, CH), :], bbuf[i], bsem[i])
        return da, db

    pend = issue_a(0, 0)
    for sub in range(NCH_A):
        i = sub % 2
        nxt = issue_a(sub + 1, 1 - i) if sub + 1 < NCH_A else None
        pend[0].wait()
        pend[1].wait()
        sums = row_sums(abuf[i], bbuf[i])

        def group_a(g, _, sub=sub, sums=sums):
            o1v = sums(g)
            w16 = swv[pl.ds(sub * CH + g * L, L)]
            accv[...] = accv[...] + (s1 * w16 * o1v) / (s1 + o1v)

        lax.fori_loop(0, NG, group_a, None)
        pend = nxt

    def issue_b(c, i):
        pltpu.async_copy(u_hbm.at[ua_all.at[pl.ds(c * CH, CH)]],
                         abuf[i], asem[i])
        pltpu.async_copy(u_hbm.at[ub_all.at[pl.ds(c * CH, CH)]],
                         bbuf[i], bsem[i])

    def wait_b(i):
        pltpu.make_async_copy(u_hbm.at[ua_all.at[pl.ds(0, CH)]],
                              abuf[i], asem[i]).wait()
        pltpu.make_async_copy(u_hbm.at[ub_all.at[pl.ds(0, CH)]],
                              bbuf[i], bsem[i]).wait()

    def compute_b(c, i):
        sums = row_sums(abuf[i], bbuf[i])

        def group_b(g, _):
            o2v = sums(g)
            pw16 = pwv[pl.ds(c * CH + g * L, L)]
            accv[...] = accv[...] + (lam * s2 * pw16 * o2v) / (s2 + o2v)

        lax.fori_loop(0, NG, group_b, None)

    issue_b(0, 0)
    issue_b(1, 1)

    def chunk_pair_b(t, _):
        c0 = 2 * t
        wait_b(0)
        compute_b(c0, 0)

        @pl.when(t < NCH_B // 2 - 1)
        def _():
            issue_b(c0 + 2, 0)

        wait_b(1)
        compute_b(c0 + 1, 1)

        @pl.when(t < NCH_B // 2 - 1)
        def _():
            issue_b(c0 + 3, 1)

    lax.fori_loop(0, NCH_B // 2, chunk_pair_b, None)

    pltpu.sync_copy(accv, out_hbm.at[pl.ds(wid * L, L)])


_loss_kernel = pl.kernel(
    _loss_body,
    out_type=jax.ShapeDtypeStruct((NW * L,), jnp.float32),
    mesh=plsc.VectorSubcoreMesh(core_axis_name="c", subcore_axis_name="s"),
    compiler_params=pltpu.CompilerParams(needs_layout_passes=False),
    scratch_types=[
        pltpu.VMEM((B,), jnp.int32),
        pltpu.VMEM((PPW,), jnp.int32),
        pltpu.VMEM((PPW,), jnp.int32),
        pltpu.VMEM((PPW,), jnp.int32),
        pltpu.VMEM((PPW,), jnp.int32),
        pltpu.VMEM((CH, NDIM), jnp.float32),
        pltpu.VMEM((CH, NDIM), jnp.float32),
        pltpu.VMEM((CH, NDIM), jnp.float32),
        pltpu.VMEM((CH, NDIM), jnp.float32),
        pltpu.VMEM((RPW,), jnp.float32),
        pltpu.VMEM((PPW,), jnp.float32),
        pltpu.VMEM((L,), jnp.float32),
        pltpu.VMEM((L,), jnp.float32),
        pltpu.VMEM((L,), jnp.float32),
        pltpu.VMEM((L,), jnp.float32),
        pltpu.SemaphoreType.DMA,
        pltpu.SemaphoreType.DMA,
        pltpu.SemaphoreType.DMA,
        pltpu.SemaphoreType.DMA,
    ],
)


def kernel(enc_out, sampweights, pairweights, pairs, index, _sigma1, _sigma2,
           _lambda, U):
    pa = pairs[:, 0].astype(jnp.int32)
    pb = pairs[:, 1].astype(jnp.int32)
    idx = index.astype(jnp.int32)
    s1v = jnp.full((L,), _sigma1, jnp.float32)
    s2v = jnp.full((L,), _sigma2, jnp.float32)
    lamv = jnp.full((L,), _lambda, jnp.float32)
    partials = _loss_kernel(U, enc_out, sampweights, pairweights, pa, pb, idx,
                            s1v, s2v, lamv)
    return jnp.sum(partials) / (enc_out.shape[0] * enc_out.shape[1])


# speedup vs baseline: 1.5447x; 1.0245x over previous
import jax
import jax.numpy as jnp
from jax import lax
from jax.experimental import pallas as pl
from jax.experimental.pallas import tpu as pltpu
from jax.experimental.pallas import tpu_sc as plsc

NSAMPLES = 100000
NDIM = 128
B = 16384
P = 131072

NC = 2
NS = 16
NW = NC * NS
L = 16

RPW = B // NW
PPW = P // NW
CH = 128
NCH_A = RPW // CH
NCH_B = PPW // CH
NG = CH // L


def _loss_body(u_hbm, enc_hbm, sw_hbm, pw_hbm, pa_hbm, pb_hbm, idx_hbm,
               s1_hbm, s2_hbm, lam_hbm, out_hbm,
               idxall, pa_v, pb_v, ua_all, ub_all, a0, a1, b0, b1,
               swv, pwv, s1v, s2v, lamv, accv, sa0, sa1, sb0, sb1):
    wid = lax.axis_index("s") * NC + lax.axis_index("c")
    rbase = wid * RPW
    pbase = wid * PPW

    d1 = pltpu.async_copy(idx_hbm, idxall, sa0)
    d2 = pltpu.async_copy(pa_hbm.at[pl.ds(pbase, PPW)], pa_v, sa1)
    d3 = pltpu.async_copy(pb_hbm.at[pl.ds(pbase, PPW)], pb_v, sb0)
    d4 = pltpu.async_copy(sw_hbm.at[pl.ds(rbase, RPW)], swv, sb1)
    d1.wait()
    d2.wait()
    d3.wait()
    d4.wait()
    d5 = pltpu.async_copy(pw_hbm.at[pl.ds(pbase, PPW)], pwv, sa0)
    d6 = pltpu.async_copy(s1_hbm, s1v, sa1)
    d7 = pltpu.async_copy(s2_hbm, s2v, sb0)
    d8 = pltpu.async_copy(lam_hbm, lamv, sb1)
    d5.wait()
    d6.wait()
    d7.wait()
    d8.wait()

    accv[...] = jnp.zeros((L,), jnp.float32)
    iota16 = lax.iota(jnp.int32, L)
    s1 = s1v[...]
    s2 = s2v[...]
    lam = lamv[...]

    abuf = (a0, a1)
    bbuf = (b0, b1)
    asem = (sa0, sa1)
    bsem = (sb0, sb1)

    def issue_a(sub, i):
        da = pltpu.async_copy(
            u_hbm.at[idxall.at[pl.ds(rbase + sub * CH, CH)]], abuf[i], asem[i])
        db = pltpu.async_copy(
            enc_hbm.at[pl.ds(rbase + sub * CH, CH), :], bbuf[i], bsem[i])
        return da, db

    pend = issue_a(0, 0)

    def translate(t, _):
        pav = pa_v[pl.ds(t * L, L)]
        ua_all[pl.ds(t * L, L)] = plsc.load_gather(idxall, [pav])
        pbv = pb_v[pl.ds(t * L, L)]
        ub_all[pl.ds(t * L, L)] = plsc.load_gather(idxall, [pbv])

    lax.fori_loop(0, PPW // L, translate, None)

    def row_sums(ar, br):
        def sums_at(g):
            def quad(q, ov):
                for r4 in range(8):
                    lane = q * 8 + r4
                    row = g * L + lane
                    acc16 = jnp.zeros((L,), jnp.float32)
                    for k in range(NDIM // L):
                        xv = ar[row, pl.ds(k * L, L)]
                        yv = br[row, pl.ds(k * L, L)]
                        df = xv - yv
                        acc16 = acc16 + df * df
                    ov = ov + jnp.where(iota16 == lane, jnp.sum(acc16), 0.0)
                return ov
            return lax.fori_loop(0, 2, quad, jnp.zeros((L,), jnp.float32))
        return sums_at

    for sub in range(NCH_A):
        i = sub % 2
        nxt = issue_a(sub + 1, 1 - i) if sub + 1 < NCH_A else None
        pend[0].wait()
        pend[1].wait()
        sums = row_sums(abuf[i], bbuf[i])

        def group_a(g, _, sub=sub, sums=sums):
            o1v = sums(g)
            w16 = swv[pl.ds(sub * CH + g * L, L)]
            accv[...] = accv[...] + (s1 * w16 * o1v) / (s1 + o1v)

        lax.fori_loop(0, NG, group_a, None)
        pend = nxt

    def issue_b(c, i):
        pltpu.async_copy(u_hbm.at[ua_all.at[pl.ds(c * CH, CH)]],
                         abuf[i], asem[i])
        pltpu.async_copy(u_hbm.at[ub_all.at[pl.ds(c * CH, CH)]],
                         bbuf[i], bsem[i])

    def wait_b(i):
        pltpu.make_async_copy(u_hbm.at[ua_all.at[pl.ds(0, CH)]],
                              abuf[i], asem[i]).wait()
        pltpu.make_async_copy(u_hbm.at[ub_all.at[pl.ds(0, CH)]],
                              bbuf[i], bsem[i]).wait()

    def compute_b(c, i):
        sums = row_sums(abuf[i], bbuf[i])

        def group_b(g, _):
            o2v = sums(g)
            pw16 = pwv[pl.ds(c * CH + g * L, L)]
            accv[...] = accv[...] + (lam * s2 * pw16 * o2v) / (s2 + o2v)

        lax.fori_loop(0, NG, group_b, None)

    issue_b(0, 0)
    issue_b(1, 1)

    def chunk_pair_b(t, _):
        c0 = 2 * t
        wait_b(0)
        compute_b(c0, 0)

        @pl.when(t < NCH_B // 2 - 1)
        def _():
            issue_b(c0 + 2, 0)

        wait_b(1)
        compute_b(c0 + 1, 1)

        @pl.when(t < NCH_B // 2 - 1)
        def _():
            issue_b(c0 + 3, 1)

    lax.fori_loop(0, NCH_B // 2, chunk_pair_b, None)

    pltpu.sync_copy(accv, out_hbm.at[pl.ds(wid * L, L)])


_loss_kernel = pl.kernel(
    _loss_body,
    out_type=jax.ShapeDtypeStruct((NW * L,), jnp.float32),
    mesh=plsc.VectorSubcoreMesh(core_axis_name="c", subcore_axis_name="s"),
    compiler_params=pltpu.CompilerParams(needs_layout_passes=False),
    scratch_types=[
        pltpu.VMEM((B,), jnp.int32),
        pltpu.VMEM((PPW,), jnp.int32),
        pltpu.VMEM((PPW,), jnp.int32),
        pltpu.VMEM((PPW,), jnp.int32),
        pltpu.VMEM((PPW,), jnp.int32),
        pltpu.VMEM((CH, NDIM), jnp.float32),
        pltpu.VMEM((CH, NDIM), jnp.float32),
        pltpu.VMEM((CH, NDIM), jnp.float32),
        pltpu.VMEM((CH, NDIM), jnp.float32),
        pltpu.VMEM((RPW,), jnp.float32),
        pltpu.VMEM((PPW,), jnp.float32),
        pltpu.VMEM((L,), jnp.float32),
        pltpu.VMEM((L,), jnp.float32),
        pltpu.VMEM((L,), jnp.float32),
        pltpu.VMEM((L,), jnp.float32),
        pltpu.SemaphoreType.DMA,
        pltpu.SemaphoreType.DMA,
        pltpu.SemaphoreType.DMA,
        pltpu.SemaphoreType.DMA,
    ],
)


def kernel(enc_out, sampweights, pairweights, pairs, index, _sigma1, _sigma2,
           _lambda, U):
    pa = pairs[:, 0].astype(jnp.int32)
    pb = pairs[:, 1].astype(jnp.int32)
    idx = index.astype(jnp.int32)
    s1v = jnp.full((L,), _sigma1, jnp.float32)
    s2v = jnp.full((L,), _sigma2, jnp.float32)
    lamv = jnp.full((L,), _lambda, jnp.float32)
    partials = _loss_kernel(U, enc_out, sampweights, pairweights, pa, pb, idx,
                            s1v, s2v, lamv)
    return jnp.sum(partials) / (enc_out.shape[0] * enc_out.shape[1])
